# Initial kernel scaffold; baseline (speedup 1.0000x reference)
import functools

import jax
import jax.numpy as jnp
from jax.experimental import pallas as pl
from jax.experimental.pallas import tpu as pltpu

_NUM_EXPERTS = 64
_TOP_K = 2
_HIDDEN = 4096
_BLOCK_T = 512


def _router_body(x_ref, w_ref, tp_ref, ti_ref, lg_ref):
    x = x_ref[...]
    w = w_ref[...]
    logits = jax.lax.dot_general(
        x, w,
        dimension_numbers=(((1,), (1,)), ((), ())),
        preferred_element_type=jnp.float32,
        precision=jax.lax.Precision.HIGHEST,
    )
    lg_ref[...] = logits

    m1 = jnp.max(logits, axis=1, keepdims=True)
    e = jnp.exp(logits - m1)
    z = jnp.sum(e, axis=1, keepdims=True)

    iota = jax.lax.broadcasted_iota(jnp.int32, logits.shape, 1)
    sentinel = jnp.int32(_NUM_EXPERTS)
    i1 = jnp.min(jnp.where(logits == m1, iota, sentinel), axis=1, keepdims=True)
    masked = jnp.where(iota == i1, -jnp.inf, logits)
    m2 = jnp.max(masked, axis=1, keepdims=True)
    i2 = jnp.min(jnp.where(masked == m2, iota, sentinel), axis=1, keepdims=True)

    p1 = jnp.exp(m1 - m1) / z
    p2 = jnp.exp(m2 - m1) / z
    tp_ref[...] = jnp.concatenate([p1, p2], axis=1)
    ti_ref[...] = jnp.concatenate([i1, i2], axis=1)


@jax.jit
def kernel(input, weight):
    n_tokens = input.shape[0]
    grid = (n_tokens // _BLOCK_T,)
    tp, ti, lg = pl.pallas_call(
        _router_body,
        grid=grid,
        in_specs=[
            pl.BlockSpec((_BLOCK_T, _HIDDEN), lambda i: (i, 0)),
            pl.BlockSpec((_NUM_EXPERTS, _HIDDEN), lambda i: (0, 0)),
        ],
        out_specs=[
            pl.BlockSpec((_BLOCK_T, _TOP_K), lambda i: (i, 0)),
            pl.BlockSpec((_BLOCK_T, _TOP_K), lambda i: (i, 0)),
            pl.BlockSpec((_BLOCK_T, _NUM_EXPERTS), lambda i: (i, 0)),
        ],
        out_shape=[
            jax.ShapeDtypeStruct((n_tokens, _TOP_K), jnp.float32),
            jax.ShapeDtypeStruct((n_tokens, _TOP_K), jnp.int32),
            jax.ShapeDtypeStruct((n_tokens, _NUM_EXPERTS), jnp.float32),
        ],
        compiler_params=pltpu.CompilerParams(
            dimension_semantics=("arbitrary",),
        ),
    )(input, weight)
    return tp, ti, lg


# speedup vs baseline: 1.3840x; 1.3840x over previous
import functools

import jax
import jax.numpy as jnp
from jax.experimental import pallas as pl
from jax.experimental.pallas import tpu as pltpu

_NUM_EXPERTS = 64
_TOP_K = 2
_HIDDEN = 4096
_BLOCK_T = 512


def _router_body(x_ref, w_ref, tp_ref, ti_ref, lg_ref):
    x = x_ref[...]
    w = w_ref[...]
    logits = jax.lax.dot_general(
        x, w,
        dimension_numbers=(((1,), (1,)), ((), ())),
        preferred_element_type=jnp.float32,
        precision=jax.lax.Precision.DEFAULT,
    )
    lg_ref[...] = logits

    m1 = jnp.max(logits, axis=1, keepdims=True)
    e = jnp.exp(logits - m1)
    z = jnp.sum(e, axis=1, keepdims=True)

    iota = jax.lax.broadcasted_iota(jnp.int32, logits.shape, 1)
    sentinel = jnp.int32(_NUM_EXPERTS)
    i1 = jnp.min(jnp.where(logits == m1, iota, sentinel), axis=1, keepdims=True)
    masked = jnp.where(iota == i1, -jnp.inf, logits)
    m2 = jnp.max(masked, axis=1, keepdims=True)
    i2 = jnp.min(jnp.where(masked == m2, iota, sentinel), axis=1, keepdims=True)

    p1 = jnp.exp(m1 - m1) / z
    p2 = jnp.exp(m2 - m1) / z
    tp_ref[...] = jnp.concatenate([p1, p2], axis=1)
    ti_ref[...] = jnp.concatenate([i1, i2], axis=1)


@jax.jit
def kernel(input, weight):
    n_tokens = input.shape[0]
    grid = (n_tokens // _BLOCK_T,)
    tp, ti, lg = pl.pallas_call(
        _router_body,
        grid=grid,
        in_specs=[
            pl.BlockSpec((_BLOCK_T, _HIDDEN), lambda i: (i, 0)),
            pl.BlockSpec((_NUM_EXPERTS, _HIDDEN), lambda i: (0, 0)),
        ],
        out_specs=[
            pl.BlockSpec((_BLOCK_T, _TOP_K), lambda i: (i, 0)),
            pl.BlockSpec((_BLOCK_T, _TOP_K), lambda i: (i, 0)),
            pl.BlockSpec((_BLOCK_T, _NUM_EXPERTS), lambda i: (i, 0)),
        ],
        out_shape=[
            jax.ShapeDtypeStruct((n_tokens, _TOP_K), jnp.float32),
            jax.ShapeDtypeStruct((n_tokens, _TOP_K), jnp.int32),
            jax.ShapeDtypeStruct((n_tokens, _NUM_EXPERTS), jnp.float32),
        ],
        compiler_params=pltpu.CompilerParams(
            dimension_semantics=("arbitrary",),
        ),
    )(input, weight)
    return tp, ti, lg


# speedup vs baseline: 1.4661x; 1.0593x over previous
import functools

import jax
import jax.numpy as jnp
from jax.experimental import pallas as pl
from jax.experimental.pallas import tpu as pltpu

_NUM_EXPERTS = 64
_TOP_K = 2
_HIDDEN = 4096
_BLOCK_T = 1024


def _router_body(x_ref, w_ref, tp_ref, ti_ref, lg_ref):
    x = x_ref[...]
    w = w_ref[...]
    logits = jax.lax.dot_general(
        x, w,
        dimension_numbers=(((1,), (1,)), ((), ())),
        preferred_element_type=jnp.float32,
        precision=jax.lax.Precision.DEFAULT,
    )
    lg_ref[...] = logits

    m1 = jnp.max(logits, axis=1, keepdims=True)
    e = jnp.exp(logits - m1)
    z = jnp.sum(e, axis=1, keepdims=True)

    iota = jax.lax.broadcasted_iota(jnp.int32, logits.shape, 1)
    sentinel = jnp.int32(_NUM_EXPERTS)
    i1 = jnp.min(jnp.where(logits == m1, iota, sentinel), axis=1, keepdims=True)
    masked = jnp.where(iota == i1, -jnp.inf, logits)
    m2 = jnp.max(masked, axis=1, keepdims=True)
    i2 = jnp.min(jnp.where(masked == m2, iota, sentinel), axis=1, keepdims=True)

    p1 = jnp.exp(m1 - m1) / z
    p2 = jnp.exp(m2 - m1) / z
    tp_ref[...] = jnp.concatenate([p1, p2], axis=1)
    ti_ref[...] = jnp.concatenate([i1, i2], axis=1)


@jax.jit
def kernel(input, weight):
    n_tokens = input.shape[0]
    grid = (n_tokens // _BLOCK_T,)
    tp, ti, lg = pl.pallas_call(
        _router_body,
        grid=grid,
        in_specs=[
            pl.BlockSpec((_BLOCK_T, _HIDDEN), lambda i: (i, 0)),
            pl.BlockSpec((_NUM_EXPERTS, _HIDDEN), lambda i: (0, 0)),
        ],
        out_specs=[
            pl.BlockSpec((_BLOCK_T, _TOP_K), lambda i: (i, 0)),
            pl.BlockSpec((_BLOCK_T, _TOP_K), lambda i: (i, 0)),
            pl.BlockSpec((_BLOCK_T, _NUM_EXPERTS), lambda i: (i, 0)),
        ],
        out_shape=[
            jax.ShapeDtypeStruct((n_tokens, _TOP_K), jnp.float32),
            jax.ShapeDtypeStruct((n_tokens, _TOP_K), jnp.int32),
            jax.ShapeDtypeStruct((n_tokens, _NUM_EXPERTS), jnp.float32),
        ],
        compiler_params=pltpu.CompilerParams(
            dimension_semantics=("arbitrary",),
        ),
    )(input, weight)
    return tp, ti, lg


# speedup vs baseline: 1.4673x; 1.0008x over previous
import functools

import jax
import jax.numpy as jnp
from jax.experimental import pallas as pl
from jax.experimental.pallas import tpu as pltpu

_NUM_EXPERTS = 64
_TOP_K = 2
_HIDDEN = 4096
_BLOCK_T = 1024
_KSPLIT = 2
_BK = _HIDDEN // _KSPLIT


def _router_body(x0_ref, x1_ref, w_ref, tp_ref, ti_ref, lg_ref):
    w = w_ref[...]
    dn = (((1,), (1,)), ((), ()))
    logits = jax.lax.dot_general(
        x0_ref[...], w[:, :_BK], dn,
        preferred_element_type=jnp.float32,
        precision=jax.lax.Precision.DEFAULT,
    )
    logits = logits + jax.lax.dot_general(
        x1_ref[...], w[:, _BK:], dn,
        preferred_element_type=jnp.float32,
        precision=jax.lax.Precision.DEFAULT,
    )
    lg_ref[...] = logits

    m1 = jnp.max(logits, axis=1, keepdims=True)
    e = jnp.exp(logits - m1)
    z = jnp.sum(e, axis=1, keepdims=True)

    iota = jax.lax.broadcasted_iota(jnp.int32, logits.shape, 1)
    sentinel = jnp.int32(_NUM_EXPERTS)
    i1 = jnp.min(jnp.where(logits == m1, iota, sentinel), axis=1, keepdims=True)
    masked = jnp.where(iota == i1, -jnp.inf, logits)
    m2 = jnp.max(masked, axis=1, keepdims=True)
    i2 = jnp.min(jnp.where(masked == m2, iota, sentinel), axis=1, keepdims=True)

    p1 = jnp.exp(m1 - m1) / z
    p2 = jnp.exp(m2 - m1) / z
    tp_ref[...] = jnp.concatenate([p1, p2], axis=1)
    ti_ref[...] = jnp.concatenate([i1, i2], axis=1)


@jax.jit
def kernel(input, weight):
    n_tokens = input.shape[0]
    grid = (n_tokens // _BLOCK_T,)
    tp, ti, lg = pl.pallas_call(
        _router_body,
        grid=grid,
        in_specs=[
            pl.BlockSpec((_BLOCK_T, _BK), lambda i: (i, 0)),
            pl.BlockSpec((_BLOCK_T, _BK), lambda i: (i, 1)),
            pl.BlockSpec((_NUM_EXPERTS, _HIDDEN), lambda i: (0, 0)),
        ],
        out_specs=[
            pl.BlockSpec((_BLOCK_T, _TOP_K), lambda i: (i, 0)),
            pl.BlockSpec((_BLOCK_T, _TOP_K), lambda i: (i, 0)),
            pl.BlockSpec((_BLOCK_T, _NUM_EXPERTS), lambda i: (i, 0)),
        ],
        out_shape=[
            jax.ShapeDtypeStruct((n_tokens, _TOP_K), jnp.float32),
            jax.ShapeDtypeStruct((n_tokens, _TOP_K), jnp.int32),
            jax.ShapeDtypeStruct((n_tokens, _NUM_EXPERTS), jnp.float32),
        ],
        compiler_params=pltpu.CompilerParams(
            dimension_semantics=("arbitrary",),
        ),
    )(input, input, weight)
    return tp, ti, lg
